# Initial kernel scaffold; baseline (speedup 1.0000x reference)
import jax
import jax.numpy as jnp
from jax.experimental import pallas as pl


def kernel(x, codebooks, codes, scales):
    raise NotImplementedError("write your pallas kernel here")


# speedup vs baseline: 431.9904x; 431.9904x over previous
import functools

import jax
import jax.numpy as jnp
from jax import lax
from jax.experimental import pallas as pl
from jax.experimental.pallas import tpu as pltpu
from jax.experimental.pallas import tpu_sc as plsc

J = 512
C = 2
K = 256
O = 4096
ROW = C * K
NTILES = 32
O_PER = O // NTILES
JC = 64
NCH = J // JC
LANES = 16


def _lut_body(xg_ref, cb_ref, lut_ref):
    lut_ref[...] = lax.dot_general(
        xg_ref[...], cb_ref[...],
        dimension_numbers=(((1,), (1,)), ((), ())),
        preferred_element_type=jnp.float32)


def _compute_lut(xg, cbf):
    return pl.pallas_call(
        _lut_body,
        out_shape=jax.ShapeDtypeStruct((J, ROW), jnp.float32),
    )(xg, cbf)


_mesh = plsc.VectorSubcoreMesh(core_axis_name="c", subcore_axis_name="s")


@functools.partial(
    pl.kernel,
    mesh=_mesh,
    compiler_params=pltpu.CompilerParams(
        needs_layout_passes=False, use_tc_tiling_on_sc=False),
    out_type=jax.ShapeDtypeStruct((O,), jnp.float32),
    scratch_types=[
        pltpu.VMEM((JC * ROW,), jnp.float32),
        pltpu.VMEM((JC * ROW,), jnp.float32),
        pltpu.VMEM((JC, 2 * O_PER), jnp.int32),
        pltpu.VMEM((JC, 2 * O_PER), jnp.int32),
        pltpu.VMEM((2 * O_PER,), jnp.float32),
        pltpu.VMEM((O_PER,), jnp.float32),
        pltpu.VMEM((O_PER,), jnp.float32),
        pltpu.SemaphoreType.DMA,
        pltpu.SemaphoreType.DMA,
        pltpu.SemaphoreType.DMA,
        pltpu.SemaphoreType.DMA,
    ],
)
def _sc_gather(lut_hbm, codes_hbm, scales_hbm, out_hbm,
               lut_v0, lut_v1, codes_v0, codes_v1, acc_v, scl_v, out_v,
               sem_l0, sem_l1, sem_c0, sem_c1):
    wid = lax.axis_index("s") * 2 + lax.axis_index("c")
    o_base = wid * O_PER
    ob2 = o_base * 2

    lut_sems = (sem_l0, sem_l1)
    code_sems = (sem_c0, sem_c1)
    lut_bufs = (lut_v0, lut_v1)
    code_bufs = (codes_v0, codes_v1)

    def start(ci, b):
        lc = pltpu.async_copy(lut_hbm.at[ci], lut_bufs[b], lut_sems[b])
        cc = pltpu.async_copy(codes_hbm.at[ci, :, pl.ds(ob2, 2 * O_PER)],
                              code_bufs[b], code_sems[b])
        return lc, cc

    pend = {0: start(0, 0), 1: start(1, 1)}
    pltpu.sync_copy(scales_hbm.at[pl.ds(o_base, O_PER)], scl_v)

    iota16 = lax.iota(jnp.int32, LANES)
    coff = (iota16 % 2) * K

    accs = tuple(jnp.zeros((LANES,), jnp.float32) for _ in range(16))
    for ci in range(NCH):
        b = ci % 2
        lc, cc = pend[b]
        lc.wait()
        cc.wait()

        def body(j, acc, _lut=lut_bufs[b], _codes=code_bufs[b]):
            base = coff + j * ROW
            new = []
            for v in range(16):
                cvec = _codes[j, pl.ds(v * LANES, LANES)]
                g = plsc.load_gather(_lut, [cvec + base])
                new.append(acc[v] + g)
            return tuple(new)

        accs = lax.fori_loop(0, JC, body, accs)
        if ci + 2 < NCH:
            pend[b] = start(ci + 2, b)

    for v in range(16):
        acc_v[pl.ds(v * LANES, LANES)] = accs[v]

    for u in range(8):
        ev = plsc.load_gather(acc_v, [iota16 * 2 + u * 32])
        od = plsc.load_gather(acc_v, [iota16 * 2 + u * 32 + 1])
        s = (ev + od) * scl_v[pl.ds(u * LANES, LANES)]
        out_v[pl.ds(u * LANES, LANES)] = s
    pltpu.sync_copy(out_v, out_hbm.at[pl.ds(o_base, O_PER)])


def kernel(x, codebooks, codes, scales):
    xg = x.reshape(J, 8)
    cbf = codebooks.reshape(C * K, 8)
    lut = _compute_lut(xg, cbf)
    lut8 = lut.reshape(NCH, JC * ROW)
    codes8 = codes.reshape(NCH, JC, 2 * O)
    scales_flat = scales.reshape(O)
    out = _sc_gather(lut8, codes8, scales_flat)
    return out.reshape(1, O)


# speedup vs baseline: 448.9366x; 1.0392x over previous
import functools

import jax
import jax.numpy as jnp
from jax import lax
from jax.experimental import pallas as pl
from jax.experimental.pallas import tpu as pltpu
from jax.experimental.pallas import tpu_sc as plsc

J = 512
C = 2
K = 256
O = 4096
ROW = C * K
NTILES = 32
O_PER = O // NTILES
JC = 64
NCH = J // JC
LANES = 16


def _lut_body(xg_ref, cb_ref, lut_ref):
    lut_ref[...] = lax.dot_general(
        xg_ref[...], cb_ref[...],
        dimension_numbers=(((1,), (1,)), ((), ())),
        preferred_element_type=jnp.float32)


def _compute_lut(xg, cbf):
    return pl.pallas_call(
        _lut_body,
        out_shape=jax.ShapeDtypeStruct((J, ROW), jnp.float32),
    )(xg, cbf)


_mesh = plsc.VectorSubcoreMesh(core_axis_name="c", subcore_axis_name="s")


@functools.partial(
    pl.kernel,
    mesh=_mesh,
    compiler_params=pltpu.CompilerParams(
        needs_layout_passes=False, use_tc_tiling_on_sc=True),
    out_type=jax.ShapeDtypeStruct((O,), jnp.float32),
    scratch_types=[
        pltpu.VMEM((JC * ROW,), jnp.float32),
        pltpu.VMEM((JC * ROW,), jnp.float32),
        pltpu.VMEM((JC, 2 * O_PER), jnp.int32),
        pltpu.VMEM((JC, 2 * O_PER), jnp.int32),
        pltpu.VMEM((2 * O_PER,), jnp.float32),
        pltpu.VMEM((O_PER,), jnp.float32),
        pltpu.VMEM((O_PER,), jnp.float32),
        pltpu.SemaphoreType.DMA,
        pltpu.SemaphoreType.DMA,
        pltpu.SemaphoreType.DMA,
        pltpu.SemaphoreType.DMA,
    ],
)
def _sc_gather(lut_hbm, codes_hbm, scales_hbm, out_hbm,
               lut_v0, lut_v1, codes_v0, codes_v1, acc_v, scl_v, out_v,
               sem_l0, sem_l1, sem_c0, sem_c1):
    wid = lax.axis_index("s") * 2 + lax.axis_index("c")
    o_base = wid * O_PER
    ob2 = o_base * 2

    lut_sems = (sem_l0, sem_l1)
    code_sems = (sem_c0, sem_c1)
    lut_bufs = (lut_v0, lut_v1)
    code_bufs = (codes_v0, codes_v1)

    def start(ci, b):
        lc = pltpu.async_copy(lut_hbm.at[ci], lut_bufs[b], lut_sems[b])
        cc = pltpu.async_copy(codes_hbm.at[ci, :, pl.ds(ob2, 2 * O_PER)],
                              code_bufs[b], code_sems[b])
        return lc, cc

    pend = {0: start(0, 0), 1: start(1, 1)}
    pltpu.sync_copy(scales_hbm.at[pl.ds(o_base, O_PER)], scl_v)

    iota16 = lax.iota(jnp.int32, LANES)
    coff = (iota16 % 2) * K

    accs = tuple(jnp.zeros((LANES,), jnp.float32) for _ in range(16))
    for ci in range(NCH):
        b = ci % 2
        lc, cc = pend[b]
        lc.wait()
        cc.wait()

        def body(j, acc, _lut=lut_bufs[b], _codes=code_bufs[b]):
            base = coff + j * ROW
            new = []
            for v in range(16):
                cvec = _codes[j, pl.ds(v * LANES, LANES)]
                g = plsc.load_gather(_lut, [cvec + base])
                new.append(acc[v] + g)
            return tuple(new)

        accs = lax.fori_loop(0, JC, body, accs)
        if ci + 2 < NCH:
            pend[b] = start(ci + 2, b)

    for v in range(16):
        acc_v[pl.ds(v * LANES, LANES)] = accs[v]

    for u in range(8):
        ev = plsc.load_gather(acc_v, [iota16 * 2 + u * 32])
        od = plsc.load_gather(acc_v, [iota16 * 2 + u * 32 + 1])
        s = (ev + od) * scl_v[pl.ds(u * LANES, LANES)]
        out_v[pl.ds(u * LANES, LANES)] = s
    pltpu.sync_copy(out_v, out_hbm.at[pl.ds(o_base, O_PER)])


def kernel(x, codebooks, codes, scales):
    xg = x.reshape(J, 8)
    cbf = codebooks.reshape(C * K, 8)
    lut = _compute_lut(xg, cbf)
    lut8 = lut.reshape(NCH, JC * ROW)
    codes8 = codes.reshape(NCH, JC, 2 * O)
    scales_flat = scales.reshape(O)
    out = _sc_gather(lut8, codes8, scales_flat)
    return out.reshape(1, O)


# speedup vs baseline: 1011.8294x; 2.2538x over previous
import functools

import jax
import jax.numpy as jnp
from jax import lax
from jax.experimental import pallas as pl
from jax.experimental.pallas import tpu as pltpu
from jax.experimental.pallas import tpu_sc as plsc

J = 512
C = 2
K = 256
O = 4096
ROW = C * K
NTILES = 32
O_PER = O // NTILES
JC = 64
NCH = J // JC
LANES = 16


def _lut_body(xg_ref, cb_ref, lut_ref):
    res = lax.dot_general(
        xg_ref[...], cb_ref[...],
        dimension_numbers=(((1,), (1,)), ((), ())),
        preferred_element_type=jnp.float32)
    lut_ref[...] = res.reshape(J * 4, 128)


def _compute_lut(xg, cbf):
    return pl.pallas_call(
        _lut_body,
        out_shape=jax.ShapeDtypeStruct((J * 4, 128), jnp.float32),
    )(xg, cbf)


_mesh = plsc.VectorSubcoreMesh(core_axis_name="c", subcore_axis_name="s")


@functools.partial(
    pl.kernel,
    mesh=_mesh,
    compiler_params=pltpu.CompilerParams(
        needs_layout_passes=False, use_tc_tiling_on_sc=False),
    out_type=jax.ShapeDtypeStruct((O,), jnp.float32),
    scratch_types=[
        pltpu.VMEM((JC * ROW,), jnp.float32),
        pltpu.VMEM((JC * ROW,), jnp.float32),
        pltpu.VMEM((JC, 2 * O_PER), jnp.int32),
        pltpu.VMEM((JC, 2 * O_PER), jnp.int32),
        pltpu.VMEM((O_PER,), jnp.float32),
        pltpu.VMEM((O_PER,), jnp.float32),
        pltpu.SemaphoreType.DMA,
        pltpu.SemaphoreType.DMA,
        pltpu.SemaphoreType.DMA,
        pltpu.SemaphoreType.DMA,
    ],
)
def _sc_gather(lut_hbm, codes_hbm, scales_hbm, out_hbm,
               lut_v0, lut_v1, codes_v0, codes_v1, scl_v, out_v,
               sem_l0, sem_l1, sem_c0, sem_c1):
    wid = lax.axis_index("s") * 2 + lax.axis_index("c")
    o_base = wid * O_PER

    lut_sems = (sem_l0, sem_l1)
    code_sems = (sem_c0, sem_c1)
    lut_bufs = (lut_v0, lut_v1)
    code_bufs = (codes_v0, codes_v1)

    def start(ci, b):
        lc = pltpu.async_copy(lut_hbm.at[pl.ds(ci * JC * ROW, JC * ROW)],
                              lut_bufs[b], lut_sems[b])

        def issue(jr, _, _ci=ci, _b=b):
            pltpu.async_copy(
                codes_hbm.at[(_ci * JC + jr) * (O // O_PER) + wid],
                code_bufs[_b].at[jr], code_sems[_b])
            return 0

        lax.fori_loop(0, JC, issue, 0)
        return lc

    def wait_codes(b):
        pltpu.make_async_copy(
            codes_hbm.at[pl.ds(0, JC), :],
            code_bufs[b], code_sems[b]).wait()

    pend = {0: start(0, 0), 1: start(1, 1)}
    pltpu.sync_copy(scales_hbm.at[pl.ds(o_base, O_PER)], scl_v)

    accs = [jnp.zeros((LANES,), jnp.float32) for _ in range(16)]
    for ci in range(NCH):
        b = ci % 2
        pend[b].wait()
        wait_codes(b)

        def body(j, acc, _lut=lut_bufs[b], _codes=code_bufs[b]):
            base0 = jnp.full((LANES,), j * ROW, jnp.int32)
            base1 = base0 + K
            new = []
            for c in range(2):
                base = base0 if c == 0 else base1
                for g in range(8):
                    cvec = _codes[j, pl.ds(c * O_PER + g * LANES, LANES)]
                    gval = plsc.load_gather(_lut, [cvec + base])
                    new.append(acc[c * 8 + g] + gval)
            return tuple(new)

        accs = lax.fori_loop(0, JC, body, tuple(accs))
        if ci + 2 < NCH:
            pend[b] = start(ci + 2, b)

    for g in range(8):
        s = (accs[g] + accs[8 + g]) * scl_v[pl.ds(g * LANES, LANES)]
        out_v[pl.ds(g * LANES, LANES)] = s
    pltpu.sync_copy(out_v, out_hbm.at[pl.ds(o_base, O_PER)])


def kernel(x, codebooks, codes, scales):
    xg = x.reshape(J, 8)
    cbf = codebooks.reshape(C * K, 8)
    lut = _compute_lut(xg, cbf).reshape(J * ROW)
    codes_sc = codes.reshape(J, 32, O_PER, C).transpose(0, 1, 3, 2)
    codes_sc = codes_sc.reshape(J * 32, C * O_PER)
    scales_flat = scales.reshape(O)
    out = _sc_gather(lut, codes_sc, scales_flat)
    return out.reshape(1, O)


# speedup vs baseline: 1378.5418x; 1.3624x over previous
import functools

import jax
import jax.numpy as jnp
from jax import lax
from jax.experimental import pallas as pl
from jax.experimental.pallas import tpu as pltpu
from jax.experimental.pallas import tpu_sc as plsc

J = 512
C = 2
K = 256
O = 4096
ROW = C * K
NTILES = 32
O_PER = O // NTILES
JC = 64
NCH = J // JC
LANES = 16
LUT_N = J * ROW


def _lut_body(xg_ref, cb_ref, lut_ref):
    res = lax.dot_general(
        xg_ref[...], cb_ref[...],
        dimension_numbers=(((1,), (1,)), ((), ())),
        preferred_element_type=jnp.float32)
    lut_ref[...] = res.reshape(J * 4, 128)


def _compute_lut(xg, cbf):
    return pl.pallas_call(
        _lut_body,
        out_shape=jax.ShapeDtypeStruct((J * 4, 128), jnp.float32),
    )(xg, cbf)


_mesh = plsc.VectorSubcoreMesh(core_axis_name="c", subcore_axis_name="s")


@functools.partial(
    pl.kernel,
    mesh=_mesh,
    compiler_params=pltpu.CompilerParams(
        needs_layout_passes=False, use_tc_tiling_on_sc=False),
    out_type=jax.ShapeDtypeStruct((O,), jnp.float32),
    scratch_types=[
        pltpu.VMEM_SHARED((LUT_N,), jnp.float32),
        pltpu.VMEM((JC * ROW,), jnp.float32),
        pltpu.VMEM((JC * ROW,), jnp.float32),
        pltpu.VMEM((JC, 2 * O_PER), jnp.int32),
        pltpu.VMEM((JC, 2 * O_PER), jnp.int32),
        pltpu.VMEM((O_PER,), jnp.float32),
        pltpu.VMEM((O_PER,), jnp.float32),
        pltpu.SemaphoreType.DMA,
        pltpu.SemaphoreType.DMA,
        pltpu.SemaphoreType.DMA,
        pltpu.SemaphoreType.DMA,
        pltpu.SemaphoreType.DMA,
    ],
)
def _sc_gather(lut_hbm, codes_hbm, scales_hbm, out_hbm,
               lut_sh, lut_v0, lut_v1, codes_v0, codes_v1, scl_v, out_v,
               sem_l0, sem_l1, sem_c0, sem_c1, sem_st):
    sid = lax.axis_index("s")
    wid = sid * 2 + lax.axis_index("c")
    o_base = wid * O_PER

    lut_sems = (sem_l0, sem_l1)
    code_sems = (sem_c0, sem_c1)
    lut_bufs = (lut_v0, lut_v1)
    code_bufs = (codes_v0, codes_v1)

    shard = LUT_N // 16
    pltpu.async_copy(lut_hbm.at[pl.ds(sid * shard, shard)],
                     lut_sh.at[pl.ds(sid * shard, shard)], sem_st).wait()
    plsc.subcore_barrier()

    def start(ci, b):
        lc = pltpu.async_copy(lut_sh.at[pl.ds(ci * JC * ROW, JC * ROW)],
                              lut_bufs[b], lut_sems[b])

        def issue(jr, _, _ci=ci, _b=b):
            pltpu.async_copy(
                codes_hbm.at[(_ci * JC + jr) * (O // O_PER) + wid],
                code_bufs[_b].at[jr], code_sems[_b])
            return 0

        lax.fori_loop(0, JC, issue, 0)
        return lc

    def wait_codes(b):
        pltpu.make_async_copy(
            codes_hbm.at[pl.ds(0, JC), :],
            code_bufs[b], code_sems[b]).wait()

    pend = {0: start(0, 0), 1: start(1, 1)}
    pltpu.sync_copy(scales_hbm.at[pl.ds(o_base, O_PER)], scl_v)

    accs = [jnp.zeros((LANES,), jnp.float32) for _ in range(16)]
    for ci in range(NCH):
        b = ci % 2
        pend[b].wait()
        wait_codes(b)

        def body(j, acc, _lut=lut_bufs[b], _codes=code_bufs[b]):
            base0 = jnp.full((LANES,), j * ROW, jnp.int32)
            base1 = base0 + K
            new = []
            for c in range(2):
                base = base0 if c == 0 else base1
                for g in range(8):
                    cvec = _codes[j, pl.ds(c * O_PER + g * LANES, LANES)]
                    gval = plsc.load_gather(_lut, [cvec + base])
                    new.append(acc[c * 8 + g] + gval)
            return tuple(new)

        accs = lax.fori_loop(0, JC, body, tuple(accs))
        if ci + 2 < NCH:
            pend[b] = start(ci + 2, b)

    for g in range(8):
        s = (accs[g] + accs[8 + g]) * scl_v[pl.ds(g * LANES, LANES)]
        out_v[pl.ds(g * LANES, LANES)] = s
    pltpu.sync_copy(out_v, out_hbm.at[pl.ds(o_base, O_PER)])


def kernel(x, codebooks, codes, scales):
    xg = x.reshape(J, 8)
    cbf = codebooks.reshape(C * K, 8)
    lut = _compute_lut(xg, cbf).reshape(LUT_N)
    codes_sc = codes.reshape(J, 32, O_PER, C).transpose(0, 1, 3, 2)
    codes_sc = codes_sc.reshape(J * 32, C * O_PER)
    scales_flat = scales.reshape(O)
    out = _sc_gather(lut, codes_sc, scales_flat)
    return out.reshape(1, O)


# speedup vs baseline: 1387.1708x; 1.0063x over previous
import functools

import jax
import jax.numpy as jnp
from jax import lax
from jax.experimental import pallas as pl
from jax.experimental.pallas import tpu as pltpu
from jax.experimental.pallas import tpu_sc as plsc

J = 512
C = 2
K = 256
O = 4096
ROW = C * K
NTILES = 32
O_PER = O // NTILES
JC = 64
NCH = J // JC
LANES = 16
LUT_N = J * ROW


def _lut_body(xg_ref, cb_ref, lut_ref):
    res = lax.dot_general(
        xg_ref[...], cb_ref[...],
        dimension_numbers=(((1,), (1,)), ((), ())),
        preferred_element_type=jnp.float32)
    lut_ref[...] = res.reshape(J * 4, 128)


def _compute_lut(xg, cbf):
    return pl.pallas_call(
        _lut_body,
        out_shape=jax.ShapeDtypeStruct((J * 4, 128), jnp.float32),
    )(xg, cbf)


_mesh = plsc.VectorSubcoreMesh(core_axis_name="c", subcore_axis_name="s")


@functools.partial(
    pl.kernel,
    mesh=_mesh,
    compiler_params=pltpu.CompilerParams(
        needs_layout_passes=False, use_tc_tiling_on_sc=False),
    out_type=jax.ShapeDtypeStruct((O,), jnp.float32),
    scratch_types=[
        pltpu.VMEM_SHARED((LUT_N,), jnp.float32),
        pltpu.VMEM((JC * ROW,), jnp.float32),
        pltpu.VMEM((JC * ROW,), jnp.float32),
        pltpu.VMEM((JC, 2 * O_PER), jnp.int32),
        pltpu.VMEM((JC, 2 * O_PER), jnp.int32),
        pltpu.VMEM((O_PER,), jnp.float32),
        pltpu.VMEM((O_PER,), jnp.float32),
        pltpu.SemaphoreType.DMA,
        pltpu.SemaphoreType.DMA,
        pltpu.SemaphoreType.DMA,
        pltpu.SemaphoreType.DMA,
        pltpu.SemaphoreType.DMA,
    ],
)
def _sc_gather(lut_hbm, codes_hbm, scales_hbm, out_hbm,
               lut_sh, lut_v0, lut_v1, codes_v0, codes_v1, scl_v, out_v,
               sem_l0, sem_l1, sem_c0, sem_c1, sem_st):
    sid = lax.axis_index("s")
    wid = sid * 2 + lax.axis_index("c")
    o_base = wid * O_PER

    lut_sems = (sem_l0, sem_l1)
    code_sems = (sem_c0, sem_c1)
    lut_bufs = (lut_v0, lut_v1)
    code_bufs = (codes_v0, codes_v1)

    shard = LUT_N // 16
    pltpu.async_copy(lut_hbm.at[pl.ds(sid * shard, shard)],
                     lut_sh.at[pl.ds(sid * shard, shard)], sem_st).wait()
    plsc.subcore_barrier()

    def start(ci, b):
        lc = pltpu.async_copy(lut_sh.at[pl.ds(ci * JC * ROW, JC * ROW)],
                              lut_bufs[b], lut_sems[b])

        def issue(jr, _, _ci=ci, _b=b):
            pltpu.async_copy(
                codes_hbm.at[(_ci * JC + jr) * (O // O_PER) + wid],
                code_bufs[_b].at[jr], code_sems[_b])
            return 0

        lax.fori_loop(0, JC, issue, 0)
        return lc

    def wait_codes(b):
        pltpu.make_async_copy(
            codes_hbm.at[pl.ds(0, JC), :],
            code_bufs[b], code_sems[b]).wait()

    pend = {0: start(0, 0), 1: start(1, 1)}
    pltpu.sync_copy(scales_hbm.at[pl.ds(o_base, O_PER)], scl_v)

    accs = [jnp.zeros((LANES,), jnp.float32) for _ in range(16)]
    for ci in range(NCH):
        b = ci % 2
        pend[b].wait()
        wait_codes(b)

        @plsc.parallel_loop(0, JC, unroll=2, carry=tuple(accs))
        def accs(j, acc, _lut=lut_bufs[b], _codes=code_bufs[b]):
            base0 = jnp.full((LANES,), j * ROW, jnp.int32)
            base1 = base0 + K
            new = []
            for c in range(2):
                base = base0 if c == 0 else base1
                for g in range(8):
                    cvec = _codes[j, pl.ds(c * O_PER + g * LANES, LANES)]
                    gval = plsc.load_gather(_lut, [cvec + base])
                    new.append(acc[c * 8 + g] + gval)
            return tuple(new)
        if ci + 2 < NCH:
            pend[b] = start(ci + 2, b)

    for g in range(8):
        s = (accs[g] + accs[8 + g]) * scl_v[pl.ds(g * LANES, LANES)]
        out_v[pl.ds(g * LANES, LANES)] = s
    pltpu.sync_copy(out_v, out_hbm.at[pl.ds(o_base, O_PER)])


def kernel(x, codebooks, codes, scales):
    xg = x.reshape(J, 8)
    cbf = codebooks.reshape(C * K, 8)
    lut = _compute_lut(xg, cbf).reshape(LUT_N)
    codes_sc = codes.reshape(J, 32, O_PER, C).transpose(0, 1, 3, 2)
    codes_sc = codes_sc.reshape(J * 32, C * O_PER)
    scales_flat = scales.reshape(O)
    out = _sc_gather(lut, codes_sc, scales_flat)
    return out.reshape(1, O)
